# baseline (speedup 1.0000x reference)
import functools

import jax
import jax.numpy as jnp
from jax import lax
from jax.experimental import pallas as pl
from jax.experimental.pallas import tpu as pltpu
from jax.experimental.pallas import tpu_sc as plsc

N = 10000
NPAD = 10240
DUMMY = N
D = 64
DH = D // 2
NS = 16
ROWS_PT = NPAD // NS
CHUNK = 128
GS = 2
GE = GS * CHUNK
NB = 2
NG = 80
EPT = NG * GE
K_HALF = 5

_sc_mesh = plsc.VectorSubcoreMesh(
    core_axis_name="c", subcore_axis_name="s", num_cores=2
)


def _mlp_body(x_ref, w1_ref, b1_ref, w2_ref, b2_ref, o_ref):
    h = jnp.dot(x_ref[...], w1_ref[...], preferred_element_type=jnp.float32)
    h = jnp.maximum(h + b1_ref[...], 0.0)
    h = jnp.dot(h, w2_ref[...], preferred_element_type=jnp.float32)
    o_ref[...] = h + b2_ref[...]


def _mlp(x_pad, W1t, b1, W2t, b2):
    blk = 1024
    grid = NPAD // blk
    return pl.pallas_call(
        _mlp_body,
        grid=(grid,),
        in_specs=[
            pl.BlockSpec((blk, 128), lambda i: (i, 0)),
            pl.BlockSpec((128, D), lambda i: (0, 0)),
            pl.BlockSpec((1, D), lambda i: (0, 0)),
            pl.BlockSpec((D, D), lambda i: (0, 0)),
            pl.BlockSpec((1, D), lambda i: (0, 0)),
        ],
        out_specs=pl.BlockSpec((blk, D), lambda i: (i, 0)),
        out_shape=jax.ShapeDtypeStruct((NPAD, D), jnp.float32),
    )(x_pad, W1t, b1, W2t, b2)


@functools.partial(
    pl.kernel,
    out_type=[
        jax.ShapeDtypeStruct((2, K_HALF, NPAD, DH), jnp.float32),
    ],
    mesh=_sc_mesh,
    scratch_types=[
        pltpu.VMEM((NG, GE), jnp.int32),
        pltpu.VMEM((NG, GE), jnp.int32),
        pltpu.VMEM((GE, DH), jnp.float32),
        pltpu.VMEM((GE, DH), jnp.float32),
        pltpu.VMEM((ROWS_PT, DH), jnp.float32),
        pltpu.VMEM((ROWS_PT, 16), jnp.float32),
        pltpu.VMEM_SHARED((NPAD, DH), jnp.float32),
        pltpu.VMEM_SHARED((NPAD, DH), jnp.float32),
        pltpu.SemaphoreType.DMA,
        pltpu.SemaphoreType.DMA,
        pltpu.SemaphoreType.DMA,
        pltpu.SemaphoreType.DMA,
    ],
    compiler_params=pltpu.CompilerParams(use_tc_tiling_on_sc=False),
)
def _sc_prop(h_hbm, src_hbm, dst_hbm, srcw_hbm, zs_hbm,
             src_v, dst_v, stage0_v, stage1_v,
             work_v, dis_v, acc_sh, u_sh, gsem0, gsem1, ssem0, ssem1):
    cid = lax.axis_index("c")
    sid = lax.axis_index("s")
    base = sid * ROWS_PT
    sl = pl.ds(base, ROWS_PT)

    stages = (stage0_v, stage1_v)
    gsems = (gsem0, gsem1)
    ssems = (ssem0, ssem1)
    uref = u_sh
    one16 = jnp.zeros((16,), jnp.float32) + 1.0
    z16 = jnp.zeros((16,), jnp.float32)

    pltpu.sync_copy(src_hbm.at[sid], src_v)
    pltpu.sync_copy(srcw_hbm.at[sid], dst_v)

    def zero_work(r, _):
        row = work_v.at[r]
        for j in range(DH // 16):
            row[pl.ds(j * 16, 16)] = z16
        return 0

    lax.fori_loop(0, ROWS_PT, zero_work, 0)
    pltpu.sync_copy(work_v, acc_sh.at[sl])

    def fill_ones(r, _):
        row = stage0_v.at[r]
        for j in range(DH // 16):
            row[pl.ds(j * 16, 16)] = one16
        return 0

    lax.fori_loop(0, GE, fill_ones, 0)
    plsc.subcore_barrier()

    def deg_pair(g2, _):
        @pl.when(g2 > 0)
        def _():
            for b in range(NB):
                pltpu.make_async_copy(
                    stage0_v, acc_sh.at[dst_v.at[0]], ssems[b]
                ).wait()

        for b in range(NB):
            pltpu.async_copy(
                stage0_v, acc_sh.at[dst_v.at[g2 * NB + b]], ssems[b], add=True
            )
        return 0

    lax.fori_loop(0, NG // NB, deg_pair, 0)
    for b in range(NB):
        pltpu.make_async_copy(stage0_v, acc_sh.at[dst_v.at[0]], ssems[b]).wait()
    plsc.subcore_barrier()

    pltpu.sync_copy(dst_hbm.at[sid], dst_v)

    pltpu.sync_copy(acc_sh.at[sl], work_v)

    def dis_body(r, _):
        deg16 = work_v.at[r][pl.ds(0, 16)]
        d = jnp.where(deg16 > 0.0, deg16, 1.0)
        s = 0.5 * (d + 1.0)
        for _ in range(12):
            s = 0.5 * (s + d / s)
        dis_v.at[r][pl.ds(0, 16)] = jnp.where(deg16 > 0.0, 1.0 / s, 0.0)
        return 0

    lax.fori_loop(0, ROWS_PT, dis_body, 0)

    lax.fori_loop(0, ROWS_PT, zero_work, 0)
    pltpu.sync_copy(work_v, acc_sh.at[sl])

    def scale_body(r, _):
        row = work_v.at[r]
        s16 = dis_v.at[r][pl.ds(0, 16)]
        for j in range(DH // 16):
            row[pl.ds(j * 16, 16)] = row[pl.ds(j * 16, 16)] * s16
        return 0

    pltpu.sync_copy(h_hbm.at[cid, sl], work_v)
    lax.fori_loop(0, ROWS_PT, scale_body, 0)
    pltpu.sync_copy(work_v, uref.at[sl])
    plsc.subcore_barrier()

    def round_body(step, _):
        def edge_group(g, _):
            @pl.when(g > 0)
            def _():
                for b in range(NB):
                    pltpu.make_async_copy(
                        stages[b], acc_sh.at[dst_v.at[0]], ssems[b]
                    ).wait()

            for b in range(NB):
                pltpu.async_copy(
                    uref.at[src_v.at[g * NB + b]], stages[b], gsems[b]
                )
            for b in range(NB):
                pltpu.make_async_copy(
                    uref.at[src_v.at[g * NB + b]], stages[b], gsems[b]
                ).wait()
                pltpu.async_copy(
                    stages[b], acc_sh.at[dst_v.at[g * NB + b]], ssems[b],
                    add=True,
                )
            return 0

        lax.fori_loop(0, NG // NB, edge_group, 0)
        for b in range(NB):
            pltpu.make_async_copy(
                stages[b], acc_sh.at[dst_v.at[0]], ssems[b]
            ).wait()
        plsc.subcore_barrier()

        pltpu.sync_copy(acc_sh.at[sl], work_v)
        lax.fori_loop(0, ROWS_PT, scale_body, 0)

        @pl.when(step % 2 == 0)
        def _():
            k = step // 2 - 1
            pltpu.sync_copy(work_v, zs_hbm.at[cid, k, sl])

        @pl.when(step < 10)
        def _():
            lax.fori_loop(0, ROWS_PT, scale_body, 0)
            pltpu.sync_copy(work_v, uref.at[sl])
            lax.fori_loop(0, ROWS_PT, zero_work, 0)
            pltpu.sync_copy(work_v, acc_sh.at[sl])

        plsc.subcore_barrier()
        return 0

    lax.fori_loop(1, 11, round_body, 0)


def _final_body(temp_ref, h_ref, zs_ref, o_ref):
    acc = temp_ref[0] * h_ref[...]
    for k in range(K_HALF):
        zk = jnp.concatenate([zs_ref[0, k], zs_ref[1, k]], axis=1)
        acc = acc + temp_ref[k + 1] * zk
    m = jnp.max(acc, axis=1, keepdims=True)
    e = jnp.exp(acc - m)
    lse = jnp.log(jnp.sum(e, axis=1, keepdims=True))
    o_ref[...] = acc - m - lse


def _final(temp, h_pad, zs):
    blk = 1000
    grid = N // blk
    return pl.pallas_call(
        _final_body,
        grid=(grid,),
        in_specs=[
            pl.BlockSpec(memory_space=pltpu.MemorySpace.SMEM),
            pl.BlockSpec((blk, D), lambda i: (i, 0)),
            pl.BlockSpec((2, K_HALF, blk, DH), lambda i: (0, 0, i, 0)),
        ],
        out_specs=pl.BlockSpec((blk, D), lambda i: (i, 0)),
        out_shape=jax.ShapeDtypeStruct((N, D), jnp.float32),
    )(temp, h_pad, zs)


def kernel(x, edge_index, W1, b1, W2, b2, temp):
    row = edge_index[0].astype(jnp.int32)
    col = edge_index[1].astype(jnp.int32)
    is_loop = row == col
    pad_e = NS * EPT - row.shape[0]

    src = jnp.pad(row, (0, pad_e)).reshape(NS, NG, GE)
    dst = jnp.pad(jnp.where(is_loop, DUMMY, col), (0, pad_e),
                  constant_values=DUMMY).reshape(NS, NG, GE)
    srcw = jnp.pad(jnp.where(is_loop, DUMMY, row), (0, pad_e),
                   constant_values=DUMMY).reshape(NS, NG, GE)

    x_pad = jnp.pad(x, ((0, NPAD - N), (0, 0)))
    h_pad = _mlp(x_pad, W1.T, b1.reshape(1, D), W2.T, b2.reshape(1, D))
    h2 = jnp.stack([h_pad[:, :DH], h_pad[:, DH:]])
    (zs,) = _sc_prop(h2, src, dst, srcw)
    return _final(temp, h_pad, zs)
